# Initial kernel scaffold; baseline (speedup 1.0000x reference)
import jax
import jax.numpy as jnp
from jax.experimental import pallas as pl


def kernel(disc, x, label, para, pe_w, pe_b, w1, g1, b1, r1, w2, g2, b2, w3, g3, b3, r3, w4, g4, b4, w5, g5, b5, l1, g6, b6, l2, g7, b7):
    raise NotImplementedError("write your pallas kernel here")


# speedup vs baseline: 10.6524x; 10.6524x over previous
import functools

import jax
import jax.numpy as jnp
from jax import lax
from jax.experimental import pallas as pl
from jax.experimental.pallas import tpu as pltpu
from jax.experimental.pallas import tpu_sc as plsc

B, N, D, K = 4, 2048, 128, 10
KPAD = 16
TN = 256
NT = N // TN
NEG = float("-inf")


def _lrelu(v):
    return jnp.where(v >= 0, v, 0.2 * v)


def _prep_body(xt_ref, dt_ref, pw_ref, pb_ref, out_ref):
    a = xt_ref[0]
    d = dt_ref[0]
    out_ref[0] = a + jnp.dot(d, pw_ref[...], preferred_element_type=jnp.float32) + pb_ref[...]


def _prep(xt, dt, pe_w, pe_b):
    return pl.pallas_call(
        _prep_body,
        grid=(B, NT),
        in_specs=[
            pl.BlockSpec((1, TN, D), lambda b, i: (b, i, 0)),
            pl.BlockSpec((1, TN, 6), lambda b, i: (b, i, 0)),
            pl.BlockSpec((6, D), lambda b, i: (0, 0)),
            pl.BlockSpec((1, D), lambda b, i: (0, 0)),
        ],
        out_specs=pl.BlockSpec((1, TN, D), lambda b, i: (b, i, 0)),
        out_shape=jax.ShapeDtypeStruct((B, N, D), jnp.float32),
    )(xt, dt, pe_w.T, pe_b.reshape(1, D))


def _topk_core(a, full, rowsq, colsq, b, gidx_ref):
    inner = -2.0 * lax.dot_general(a, full, (((1,), (1,)), ((), ())),
                                   preferred_element_type=jnp.float32)
    pd = (-rowsq[:, None] - inner) - colsq[None, :]

    base = b * N
    colio = lax.broadcasted_iota(jnp.int32, (TN, N), 1)
    rowio = lax.broadcasted_iota(jnp.int32, (KPAD, TN), 0)
    acc = jnp.full((KPAD, TN), base, jnp.int32)
    vals = pd
    for j in range(K):
        m = jnp.max(vals, axis=1)
        cand = jnp.where(vals == m[:, None], colio, N)
        am = jnp.min(cand, axis=1)
        acc = jnp.where(rowio == j, (am + base)[None, :], acc)
        vals = jnp.where(colio == am[:, None], NEG, vals)
    gidx_ref[0] = acc


def _topk_l1_body(xtile_ref, xfull_ref, xotile_ref, xofull_ref, gidx_ref):
    ot = xotile_ref[0]
    of = xofull_ref[0]
    rowsq = jnp.sum(ot * ot, axis=0)
    colsq = jnp.sum(of * of, axis=0)
    _topk_core(xtile_ref[0], xfull_ref[0], rowsq, colsq, pl.program_id(0),
               gidx_ref)


def _topk_l1(xt, xorig):
    return pl.pallas_call(
        _topk_l1_body,
        grid=(B, NT),
        in_specs=[
            pl.BlockSpec((1, TN, D), lambda b, i: (b, i, 0)),
            pl.BlockSpec((1, N, D), lambda b, i: (b, 0, 0)),
            pl.BlockSpec((1, D, TN), lambda b, i: (b, 0, i)),
            pl.BlockSpec((1, D, N), lambda b, i: (b, 0, 0)),
        ],
        out_specs=pl.BlockSpec((1, KPAD, TN), lambda b, i: (b, 0, i)),
        out_shape=jax.ShapeDtypeStruct((B, KPAD, N), jnp.int32),
    )(xt, xt, xorig, xorig)


def _topk_ext_body(xtile_ref, xfull_ref, xxtile_ref, xxfull_ref, gidx_ref):
    rowsq = xxtile_ref[0, 0]
    colsq = xxfull_ref[0, 0]
    _topk_core(xtile_ref[0], xfull_ref[0], rowsq, colsq, pl.program_id(0),
               gidx_ref)


def _topk_ext(xt, xx):
    xx3 = xx.reshape(B, 1, N)
    return pl.pallas_call(
        _topk_ext_body,
        grid=(B, NT),
        in_specs=[
            pl.BlockSpec((1, TN, D), lambda b, i: (b, i, 0)),
            pl.BlockSpec((1, N, D), lambda b, i: (b, 0, 0)),
            pl.BlockSpec((1, 1, TN), lambda b, i: (b, 0, i)),
            pl.BlockSpec((1, 1, N), lambda b, i: (b, 0, 0)),
        ],
        out_specs=pl.BlockSpec((1, KPAD, TN), lambda b, i: (b, 0, i)),
        out_shape=jax.ShapeDtypeStruct((B, KPAD, N), jnp.int32),
    )(xt, xt, xx3, xx3)


@functools.lru_cache(maxsize=None)
def _make_sc_gather(c):
    info = plsc.get_sparse_core_info()
    nw = info.num_cores * info.num_subcores
    chunk = 128
    npairs = B * K
    total = npairs * (N // chunk)
    per_w = (total + nw - 1) // nw
    mesh = plsc.VectorSubcoreMesh(core_axis_name="c", subcore_axis_name="s")

    @functools.partial(
        pl.kernel,
        mesh=mesh,
        out_type=jax.ShapeDtypeStruct((B * K * N, c), jnp.float32),
        scratch_types=[
            pltpu.VMEM((chunk,), jnp.int32),
            pltpu.VMEM((chunk, c), jnp.float32),
            pltpu.SemaphoreType.DMA,
        ],
    )
    def k(tab_hbm, idx_hbm, out_hbm, idx_v, rows_v, sem):
        wid = lax.axis_index("c") * info.num_subcores + lax.axis_index("s")
        for i in range(per_w):
            cid = wid * per_w + i
            @pl.when(cid < total)
            def _():
                pair = cid // (N // chunk)
                n0 = (cid % (N // chunk)) * chunk
                bb = pair // K
                kk = pair % K
                src = (bb * KPAD + kk) * N + n0
                dst = pair * N + n0
                pltpu.sync_copy(idx_hbm.at[pl.ds(src, chunk)], idx_v)
                pltpu.async_copy(tab_hbm.at[idx_v], rows_v, sem).wait()
                pltpu.sync_copy(rows_v, out_hbm.at[pl.ds(dst, chunk)])

    return k


def _gather_rows(tab_flat, gidx_flat, c):
    return _make_sc_gather(c)(tab_flat, gidx_flat)


def _gf_tile(g_ref, xt_ref, ps_ref):
    cen = xt_ref[0]
    feat = g_ref[0]
    p0 = ps_ref[0]
    p1 = ps_ref[1]
    ga = (feat - cen[None, :, :]) * p0
    gb = jnp.broadcast_to((cen * p1)[None, :, :], feat.shape)
    return jnp.concatenate([ga, gb], axis=2).reshape(K * feat.shape[1], 2 * feat.shape[2])


def _conv_res_body(g_ref, xt_ref, wt_ref, rt_ref, ps_ref, y_ref, r_ref):
    gf = _gf_tile(g_ref, xt_ref, ps_ref)
    o = wt_ref.shape[1]
    y = jnp.dot(gf, wt_ref[...], preferred_element_type=jnp.float32)
    r = jnp.dot(gf, rt_ref[...], preferred_element_type=jnp.float32)
    y_ref[0, :, :, :] = y.reshape(K, TN, o)
    r_ref[0, :, :, :] = r.reshape(K, TN, o)


def _conv_res(g4, xt, wt, rt, ps):
    o = wt.shape[1]
    return pl.pallas_call(
        _conv_res_body,
        grid=(B, NT),
        in_specs=[
            pl.BlockSpec((1, K, TN, D), lambda b, i: (b, 0, i, 0)),
            pl.BlockSpec((1, TN, D), lambda b, i: (b, i, 0)),
            pl.BlockSpec((2 * D, o), lambda b, i: (0, 0)),
            pl.BlockSpec((2 * D, o), lambda b, i: (0, 0)),
            pl.BlockSpec(memory_space=pltpu.SMEM),
        ],
        out_specs=[
            pl.BlockSpec((1, K, TN, o), lambda b, i: (b, 0, i, 0)),
            pl.BlockSpec((1, K, TN, o), lambda b, i: (b, 0, i, 0)),
        ],
        out_shape=[
            jax.ShapeDtypeStruct((B, K, N, o), jnp.float32),
            jax.ShapeDtypeStruct((B, K, N, o), jnp.float32),
        ],
    )(g4, xt, wt, rt, ps)


def _conv_nores_body(g_ref, xt_ref, wt_ref, ps_ref, y_ref):
    gf = _gf_tile(g_ref, xt_ref, ps_ref)
    o = wt_ref.shape[1]
    y = jnp.dot(gf, wt_ref[...], preferred_element_type=jnp.float32)
    y_ref[0, :, :, :] = y.reshape(K, TN, o)


def _conv_nores(g4, xt, wt, ps):
    o = wt.shape[1]
    return pl.pallas_call(
        _conv_nores_body,
        grid=(B, NT),
        in_specs=[
            pl.BlockSpec((1, K, TN, D), lambda b, i: (b, 0, i, 0)),
            pl.BlockSpec((1, TN, D), lambda b, i: (b, i, 0)),
            pl.BlockSpec((2 * D, o), lambda b, i: (0, 0)),
            pl.BlockSpec(memory_space=pltpu.SMEM),
        ],
        out_specs=pl.BlockSpec((1, K, TN, o), lambda b, i: (b, 0, i, 0)),
        out_shape=jax.ShapeDtypeStruct((B, K, N, o), jnp.float32),
    )(g4, xt, wt, ps)


def _apply_body(y_ref, r_ref, m_ref, v_ref, gv_ref, bv_ref, out_ref):
    a = _lrelu((y_ref[0] - m_ref[...]) / jnp.sqrt(v_ref[...] + 1e-5)
               * gv_ref[...] + bv_ref[...])
    if r_ref is not None:
        a = a + r_ref[0]
    out_ref[0] = jnp.max(a, axis=0)


def _apply(y4, r4, m, v, gv, bv):
    o = y4.shape[3]
    body = _apply_body if r4 is not None else (
        lambda y_ref, m_ref, v_ref, gv_ref, bv_ref, out_ref:
        _apply_body(y_ref, None, m_ref, v_ref, gv_ref, bv_ref, out_ref))
    vec = pl.BlockSpec((1, o), lambda b, i: (0, 0))
    big = pl.BlockSpec((1, K, TN, o), lambda b, i: (b, 0, i, 0))
    specs = [big] + ([big] if r4 is not None else []) + [vec, vec, vec, vec]
    args = [y4] + ([r4] if r4 is not None else []) + [
        m.reshape(1, o), v.reshape(1, o), gv.reshape(1, o), bv.reshape(1, o)]
    return pl.pallas_call(
        body,
        grid=(B, NT),
        in_specs=specs,
        out_specs=pl.BlockSpec((1, TN, o), lambda b, i: (b, i, 0)),
        out_shape=jax.ShapeDtypeStruct((B, N, o), jnp.float32),
    )(*args)


def _edge_layer(xt, w, r, ps, gv, bv, xx=None):
    if xx is None:
        gidx = _topk_l1(xt, xt.transpose(0, 2, 1))
    else:
        gidx = _topk_ext(xt, xx)
    g = _gather_rows(xt.reshape(B * N, D), gidx.reshape(B * KPAD * N), D)
    g4 = g.reshape(B, K, N, D)
    wt = w.T
    if r is not None:
        y4, r4 = _conv_res(g4, xt, wt, r.T, ps)
    else:
        y4 = _conv_nores(g4, xt, wt, ps)
        r4 = None
    m = jnp.mean(y4, axis=(0, 1, 2))
    v = jnp.var(y4, axis=(0, 1, 2))
    return _apply(y4, r4, m, v, gv, bv)


def _y5_body(x1_ref, x2_ref, x3_ref, x4_ref, wt_ref, y_ref):
    xc = jnp.concatenate([x1_ref[0], x2_ref[0], x3_ref[0], x4_ref[0]], axis=1)
    y_ref[0] = jnp.dot(xc, wt_ref[...], preferred_element_type=jnp.float32)


def _y5(x1, x2, x3, x4, w5):
    return pl.pallas_call(
        _y5_body,
        grid=(B, NT),
        in_specs=[
            pl.BlockSpec((1, TN, 128), lambda b, i: (b, i, 0)),
            pl.BlockSpec((1, TN, 128), lambda b, i: (b, i, 0)),
            pl.BlockSpec((1, TN, 128), lambda b, i: (b, i, 0)),
            pl.BlockSpec((1, TN, 256), lambda b, i: (b, i, 0)),
            pl.BlockSpec((640, 1024), lambda b, i: (0, 0)),
        ],
        out_specs=pl.BlockSpec((1, TN, 1024), lambda b, i: (b, i, 0)),
        out_shape=jax.ShapeDtypeStruct((B, N, 1024), jnp.float32),
    )(x1, x2, x3, x4, w5.T)


def _pool_body(y_ref, m_ref, v_ref, gv_ref, bv_ref, mx_ref, sm_ref):
    v = _lrelu((y_ref[0] - m_ref[...]) / jnp.sqrt(v_ref[...] + 1e-5)
               * gv_ref[...] + bv_ref[...])
    pm = jnp.max(v, axis=0, keepdims=True)
    ps = jnp.sum(v, axis=0, keepdims=True)

    @pl.when(pl.program_id(1) == 0)
    def _():
        mx_ref[0] = jnp.full_like(mx_ref[0], NEG)
        sm_ref[0] = jnp.zeros_like(sm_ref[0])

    mx_ref[0] = jnp.maximum(mx_ref[0], pm)
    sm_ref[0] += ps


def _pool(y5, m5, v5, g5, b5):
    return pl.pallas_call(
        _pool_body,
        grid=(B, NT),
        in_specs=[
            pl.BlockSpec((1, TN, 1024), lambda b, i: (b, i, 0)),
            pl.BlockSpec((1, 1024), lambda b, i: (0, 0)),
            pl.BlockSpec((1, 1024), lambda b, i: (0, 0)),
            pl.BlockSpec((1, 1024), lambda b, i: (0, 0)),
            pl.BlockSpec((1, 1024), lambda b, i: (0, 0)),
        ],
        out_specs=[
            pl.BlockSpec((1, 1, 1024), lambda b, i: (b, 0, 0)),
            pl.BlockSpec((1, 1, 1024), lambda b, i: (b, 0, 0)),
        ],
        out_shape=[
            jax.ShapeDtypeStruct((B, 1, 1024), jnp.float32),
            jax.ShapeDtypeStruct((B, 1, 1024), jnp.float32),
        ],
    )(y5, m5.reshape(1, 1024), v5.reshape(1, 1024),
      g5.reshape(1, 1024), b5.reshape(1, 1024))


def _head_body(mx_ref, sm_ref, l1t_ref, l2t_ref, g6_ref, b6_ref, g7_ref,
               b7_ref, out_ref):
    xf = jnp.concatenate([mx_ref[:, 0, :], sm_ref[:, 0, :] / float(N)], axis=1)
    a = jnp.dot(xf, l1t_ref[...], preferred_element_type=jnp.float32)
    m = jnp.mean(a, axis=0, keepdims=True)
    v = jnp.mean((a - m) * (a - m), axis=0, keepdims=True)
    h = _lrelu((a - m) / jnp.sqrt(v + 1e-5) * g6_ref[...] + b6_ref[...])
    a2 = jnp.dot(h, l2t_ref[...], preferred_element_type=jnp.float32)
    m2 = jnp.mean(a2, axis=0, keepdims=True)
    v2 = jnp.mean((a2 - m2) * (a2 - m2), axis=0, keepdims=True)
    out_ref[...] = _lrelu((a2 - m2) / jnp.sqrt(v2 + 1e-5) * g7_ref[...] + b7_ref[...])


def _head(mx, sm, l1, l2, g6, b6, g7, b7):
    return pl.pallas_call(
        _head_body,
        out_shape=jax.ShapeDtypeStruct((B, 256), jnp.float32),
    )(mx, sm, l1.T, l2.T,
      g6.reshape(1, 512), b6.reshape(1, 512),
      g7.reshape(1, 256), b7.reshape(1, 256))


def kernel(disc, x, label, para, pe_w, pe_b, w1, g1, b1, r1, w2, g2, b2,
           w3, g3, b3, r3, w4, g4, b4, w5, g5, b5, l1, g6, b6, l2, g7, b7):
    xt0 = _prep(x.transpose(0, 2, 1), disc.transpose(0, 2, 1), pe_w, pe_b)

    x1 = _edge_layer(xt0, w1, r1, para[0], g1, b1)
    x2 = _edge_layer(x1, w2, None, para[2], g2, b2, xx=jnp.sum(x1 * x1, axis=2))
    x3 = _edge_layer(x2, w3, r3, para[4], g3, b3, xx=jnp.sum(x2 * x2, axis=2))
    x4 = _edge_layer(x3, w4, None, para[6], g4, b4, xx=jnp.sum(x3 * x3, axis=2))

    y5 = _y5(x1, x2, x3, x4, w5)
    m5 = jnp.mean(y5, axis=(0, 1))
    v5 = jnp.var(y5, axis=(0, 1))
    mx, sm = _pool(y5, m5, v5, g5, b5)
    return _head(mx, sm, l1, l2, g6, b6, g7, b7)


# speedup vs baseline: 11.0998x; 1.0420x over previous
import functools

import jax
import jax.numpy as jnp
from jax import lax
from jax.experimental import pallas as pl
from jax.experimental.pallas import tpu as pltpu
from jax.experimental.pallas import tpu_sc as plsc

B, N, D, K = 4, 2048, 128, 10
KPAD = 16
TN = 256
NT = N // TN
TNK = 512
NTK = N // TNK
NEG = float("-inf")


def _lrelu(v):
    return jnp.where(v >= 0, v, 0.2 * v)


def _prep_body(xt_ref, dt_ref, pw_ref, pb_ref, out_ref):
    a = xt_ref[0]
    d = dt_ref[0]
    out_ref[0] = a + jnp.dot(d, pw_ref[...], preferred_element_type=jnp.float32) + pb_ref[...]


def _prep(xt, dt, pe_w, pe_b):
    return pl.pallas_call(
        _prep_body,
        grid=(B, NT),
        in_specs=[
            pl.BlockSpec((1, TN, D), lambda b, i: (b, i, 0)),
            pl.BlockSpec((1, TN, 6), lambda b, i: (b, i, 0)),
            pl.BlockSpec((6, D), lambda b, i: (0, 0)),
            pl.BlockSpec((1, D), lambda b, i: (0, 0)),
        ],
        out_specs=pl.BlockSpec((1, TN, D), lambda b, i: (b, i, 0)),
        out_shape=jax.ShapeDtypeStruct((B, N, D), jnp.float32),
    )(xt, dt, pe_w.T, pe_b.reshape(1, D))


def _topk_core(a, full, rowsq, colsq, b, gidx_ref):
    tn = a.shape[0]
    inner = -2.0 * lax.dot_general(a, full, (((1,), (1,)), ((), ())),
                                   preferred_element_type=jnp.float32)
    pd = (-rowsq[:, None] - inner) - colsq[None, :]

    base = b * N
    colio = lax.broadcasted_iota(jnp.int32, (tn, N), 1)
    rowio = lax.broadcasted_iota(jnp.int32, (KPAD, tn), 0)
    acc = jnp.full((KPAD, tn), base, jnp.int32)
    vals = pd
    for j in range(K):
        m = jnp.max(vals, axis=1)
        cand = jnp.where(vals == m[:, None], colio, N)
        am = jnp.min(cand, axis=1)
        acc = jnp.where(rowio == j, (am + base)[None, :], acc)
        vals = jnp.where(colio == am[:, None], NEG, vals)
    gidx_ref[0] = acc


def _topk_l1_body(xtile_ref, xfull_ref, xotile_ref, xofull_ref, gidx_ref):
    ot = xotile_ref[0]
    of = xofull_ref[0]
    rowsq = jnp.sum(ot * ot, axis=0)
    colsq = jnp.sum(of * of, axis=0)
    _topk_core(xtile_ref[0], xfull_ref[0], rowsq, colsq, pl.program_id(0),
               gidx_ref)


def _topk_l1(xt, xorig):
    return pl.pallas_call(
        _topk_l1_body,
        grid=(B, NTK),
        in_specs=[
            pl.BlockSpec((1, TNK, D), lambda b, i: (b, i, 0)),
            pl.BlockSpec((1, N, D), lambda b, i: (b, 0, 0)),
            pl.BlockSpec((1, D, TNK), lambda b, i: (b, 0, i)),
            pl.BlockSpec((1, D, N), lambda b, i: (b, 0, 0)),
        ],
        out_specs=pl.BlockSpec((1, KPAD, TNK), lambda b, i: (b, 0, i)),
        out_shape=jax.ShapeDtypeStruct((B, KPAD, N), jnp.int32),
    )(xt, xt, xorig, xorig)


def _topk_ext_body(xtile_ref, xfull_ref, xxtile_ref, xxfull_ref, gidx_ref):
    rowsq = xxtile_ref[0, 0]
    colsq = xxfull_ref[0, 0]
    _topk_core(xtile_ref[0], xfull_ref[0], rowsq, colsq, pl.program_id(0),
               gidx_ref)


def _topk_ext(xt, xx):
    xx3 = xx.reshape(B, 1, N)
    return pl.pallas_call(
        _topk_ext_body,
        grid=(B, NTK),
        in_specs=[
            pl.BlockSpec((1, TNK, D), lambda b, i: (b, i, 0)),
            pl.BlockSpec((1, N, D), lambda b, i: (b, 0, 0)),
            pl.BlockSpec((1, 1, TNK), lambda b, i: (b, 0, i)),
            pl.BlockSpec((1, 1, N), lambda b, i: (b, 0, 0)),
        ],
        out_specs=pl.BlockSpec((1, KPAD, TNK), lambda b, i: (b, 0, i)),
        out_shape=jax.ShapeDtypeStruct((B, KPAD, N), jnp.int32),
    )(xt, xt, xx3, xx3)


@functools.lru_cache(maxsize=None)
def _make_sc_gather(c):
    info = plsc.get_sparse_core_info()
    nw = info.num_cores * info.num_subcores
    chunk = 128
    npairs = B * K
    total = npairs * (N // chunk)
    per_w = (total + nw - 1) // nw
    mesh = plsc.VectorSubcoreMesh(core_axis_name="c", subcore_axis_name="s")

    @functools.partial(
        pl.kernel,
        mesh=mesh,
        out_type=jax.ShapeDtypeStruct((B * K * N, c), jnp.float32),
        scratch_types=[
            pltpu.VMEM((chunk,), jnp.int32),
            pltpu.VMEM((chunk, c), jnp.float32),
            pltpu.SemaphoreType.DMA,
        ],
    )
    def k(tab_hbm, idx_hbm, out_hbm, idx_v, rows_v, sem):
        wid = lax.axis_index("c") * info.num_subcores + lax.axis_index("s")
        for i in range(per_w):
            cid = wid * per_w + i
            @pl.when(cid < total)
            def _():
                pair = cid // (N // chunk)
                n0 = (cid % (N // chunk)) * chunk
                bb = pair // K
                kk = pair % K
                src = (bb * KPAD + kk) * N + n0
                dst = pair * N + n0
                pltpu.sync_copy(idx_hbm.at[pl.ds(src, chunk)], idx_v)
                pltpu.async_copy(tab_hbm.at[idx_v], rows_v, sem).wait()
                pltpu.sync_copy(rows_v, out_hbm.at[pl.ds(dst, chunk)])

    return k


def _gather_rows(tab_flat, gidx_flat, c):
    return _make_sc_gather(c)(tab_flat, gidx_flat)


def _gf_tile(g_ref, xt_ref, ps_ref):
    cen = xt_ref[0]
    feat = g_ref[0]
    p0 = ps_ref[0]
    p1 = ps_ref[1]
    ga = (feat - cen[None, :, :]) * p0
    gb = jnp.broadcast_to((cen * p1)[None, :, :], feat.shape)
    return jnp.concatenate([ga, gb], axis=2).reshape(K * feat.shape[1], 2 * feat.shape[2])


def _conv_res_body(g_ref, xt_ref, wt_ref, rt_ref, ps_ref, y_ref, r_ref):
    gf = _gf_tile(g_ref, xt_ref, ps_ref)
    o = wt_ref.shape[1]
    y = jnp.dot(gf, wt_ref[...], preferred_element_type=jnp.float32)
    r = jnp.dot(gf, rt_ref[...], preferred_element_type=jnp.float32)
    y_ref[0, :, :, :] = y.reshape(K, TN, o)
    r_ref[0, :, :, :] = r.reshape(K, TN, o)


def _conv_res(g4, xt, wt, rt, ps):
    o = wt.shape[1]
    return pl.pallas_call(
        _conv_res_body,
        grid=(B, NT),
        in_specs=[
            pl.BlockSpec((1, K, TN, D), lambda b, i: (b, 0, i, 0)),
            pl.BlockSpec((1, TN, D), lambda b, i: (b, i, 0)),
            pl.BlockSpec((2 * D, o), lambda b, i: (0, 0)),
            pl.BlockSpec((2 * D, o), lambda b, i: (0, 0)),
            pl.BlockSpec(memory_space=pltpu.SMEM),
        ],
        out_specs=[
            pl.BlockSpec((1, K, TN, o), lambda b, i: (b, 0, i, 0)),
            pl.BlockSpec((1, K, TN, o), lambda b, i: (b, 0, i, 0)),
        ],
        out_shape=[
            jax.ShapeDtypeStruct((B, K, N, o), jnp.float32),
            jax.ShapeDtypeStruct((B, K, N, o), jnp.float32),
        ],
    )(g4, xt, wt, rt, ps)


def _conv_nores_body(g_ref, xt_ref, wt_ref, ps_ref, y_ref):
    gf = _gf_tile(g_ref, xt_ref, ps_ref)
    o = wt_ref.shape[1]
    y = jnp.dot(gf, wt_ref[...], preferred_element_type=jnp.float32)
    y_ref[0, :, :, :] = y.reshape(K, TN, o)


def _conv_nores(g4, xt, wt, ps):
    o = wt.shape[1]
    return pl.pallas_call(
        _conv_nores_body,
        grid=(B, NT),
        in_specs=[
            pl.BlockSpec((1, K, TN, D), lambda b, i: (b, 0, i, 0)),
            pl.BlockSpec((1, TN, D), lambda b, i: (b, i, 0)),
            pl.BlockSpec((2 * D, o), lambda b, i: (0, 0)),
            pl.BlockSpec(memory_space=pltpu.SMEM),
        ],
        out_specs=pl.BlockSpec((1, K, TN, o), lambda b, i: (b, 0, i, 0)),
        out_shape=jax.ShapeDtypeStruct((B, K, N, o), jnp.float32),
    )(g4, xt, wt, ps)


def _apply_body(y_ref, r_ref, m_ref, v_ref, gv_ref, bv_ref, out_ref):
    a = _lrelu((y_ref[0] - m_ref[...]) / jnp.sqrt(v_ref[...] + 1e-5)
               * gv_ref[...] + bv_ref[...])
    if r_ref is not None:
        a = a + r_ref[0]
    out_ref[0] = jnp.max(a, axis=0)


def _apply(y4, r4, m, v, gv, bv):
    o = y4.shape[3]
    body = _apply_body if r4 is not None else (
        lambda y_ref, m_ref, v_ref, gv_ref, bv_ref, out_ref:
        _apply_body(y_ref, None, m_ref, v_ref, gv_ref, bv_ref, out_ref))
    vec = pl.BlockSpec((1, o), lambda b, i: (0, 0))
    big = pl.BlockSpec((1, K, TN, o), lambda b, i: (b, 0, i, 0))
    specs = [big] + ([big] if r4 is not None else []) + [vec, vec, vec, vec]
    args = [y4] + ([r4] if r4 is not None else []) + [
        m.reshape(1, o), v.reshape(1, o), gv.reshape(1, o), bv.reshape(1, o)]
    return pl.pallas_call(
        body,
        grid=(B, NT),
        in_specs=specs,
        out_specs=pl.BlockSpec((1, TN, o), lambda b, i: (b, i, 0)),
        out_shape=jax.ShapeDtypeStruct((B, N, o), jnp.float32),
    )(*args)


def _edge_layer(xt, w, r, ps, gv, bv, xx=None):
    if xx is None:
        gidx = _topk_l1(xt, xt.transpose(0, 2, 1))
    else:
        gidx = _topk_ext(xt, xx)
    g = _gather_rows(xt.reshape(B * N, D), gidx.reshape(B * KPAD * N), D)
    g4 = g.reshape(B, K, N, D)
    wt = w.T
    if r is not None:
        y4, r4 = _conv_res(g4, xt, wt, r.T, ps)
    else:
        y4 = _conv_nores(g4, xt, wt, ps)
        r4 = None
    m = jnp.mean(y4, axis=(0, 1, 2))
    v = jnp.var(y4, axis=(0, 1, 2))
    return _apply(y4, r4, m, v, gv, bv)


def _y5_body(x1_ref, x2_ref, x3_ref, x4_ref, wt_ref, y_ref):
    xc = jnp.concatenate([x1_ref[0], x2_ref[0], x3_ref[0], x4_ref[0]], axis=1)
    y_ref[0] = jnp.dot(xc, wt_ref[...], preferred_element_type=jnp.float32)


def _y5(x1, x2, x3, x4, w5):
    return pl.pallas_call(
        _y5_body,
        grid=(B, NT),
        in_specs=[
            pl.BlockSpec((1, TN, 128), lambda b, i: (b, i, 0)),
            pl.BlockSpec((1, TN, 128), lambda b, i: (b, i, 0)),
            pl.BlockSpec((1, TN, 128), lambda b, i: (b, i, 0)),
            pl.BlockSpec((1, TN, 256), lambda b, i: (b, i, 0)),
            pl.BlockSpec((640, 1024), lambda b, i: (0, 0)),
        ],
        out_specs=pl.BlockSpec((1, TN, 1024), lambda b, i: (b, i, 0)),
        out_shape=jax.ShapeDtypeStruct((B, N, 1024), jnp.float32),
    )(x1, x2, x3, x4, w5.T)


def _pool_body(y_ref, m_ref, v_ref, gv_ref, bv_ref, mx_ref, sm_ref):
    v = _lrelu((y_ref[0] - m_ref[...]) / jnp.sqrt(v_ref[...] + 1e-5)
               * gv_ref[...] + bv_ref[...])
    pm = jnp.max(v, axis=0, keepdims=True)
    ps = jnp.sum(v, axis=0, keepdims=True)

    @pl.when(pl.program_id(1) == 0)
    def _():
        mx_ref[0] = jnp.full_like(mx_ref[0], NEG)
        sm_ref[0] = jnp.zeros_like(sm_ref[0])

    mx_ref[0] = jnp.maximum(mx_ref[0], pm)
    sm_ref[0] += ps


def _pool(y5, m5, v5, g5, b5):
    return pl.pallas_call(
        _pool_body,
        grid=(B, NT),
        in_specs=[
            pl.BlockSpec((1, TN, 1024), lambda b, i: (b, i, 0)),
            pl.BlockSpec((1, 1024), lambda b, i: (0, 0)),
            pl.BlockSpec((1, 1024), lambda b, i: (0, 0)),
            pl.BlockSpec((1, 1024), lambda b, i: (0, 0)),
            pl.BlockSpec((1, 1024), lambda b, i: (0, 0)),
        ],
        out_specs=[
            pl.BlockSpec((1, 1, 1024), lambda b, i: (b, 0, 0)),
            pl.BlockSpec((1, 1, 1024), lambda b, i: (b, 0, 0)),
        ],
        out_shape=[
            jax.ShapeDtypeStruct((B, 1, 1024), jnp.float32),
            jax.ShapeDtypeStruct((B, 1, 1024), jnp.float32),
        ],
    )(y5, m5.reshape(1, 1024), v5.reshape(1, 1024),
      g5.reshape(1, 1024), b5.reshape(1, 1024))


def _head_body(mx_ref, sm_ref, l1t_ref, l2t_ref, g6_ref, b6_ref, g7_ref,
               b7_ref, out_ref):
    xf = jnp.concatenate([mx_ref[:, 0, :], sm_ref[:, 0, :] / float(N)], axis=1)
    a = jnp.dot(xf, l1t_ref[...], preferred_element_type=jnp.float32)
    m = jnp.mean(a, axis=0, keepdims=True)
    v = jnp.mean((a - m) * (a - m), axis=0, keepdims=True)
    h = _lrelu((a - m) / jnp.sqrt(v + 1e-5) * g6_ref[...] + b6_ref[...])
    a2 = jnp.dot(h, l2t_ref[...], preferred_element_type=jnp.float32)
    m2 = jnp.mean(a2, axis=0, keepdims=True)
    v2 = jnp.mean((a2 - m2) * (a2 - m2), axis=0, keepdims=True)
    out_ref[...] = _lrelu((a2 - m2) / jnp.sqrt(v2 + 1e-5) * g7_ref[...] + b7_ref[...])


def _head(mx, sm, l1, l2, g6, b6, g7, b7):
    return pl.pallas_call(
        _head_body,
        out_shape=jax.ShapeDtypeStruct((B, 256), jnp.float32),
    )(mx, sm, l1.T, l2.T,
      g6.reshape(1, 512), b6.reshape(1, 512),
      g7.reshape(1, 256), b7.reshape(1, 256))


def kernel(disc, x, label, para, pe_w, pe_b, w1, g1, b1, r1, w2, g2, b2,
           w3, g3, b3, r3, w4, g4, b4, w5, g5, b5, l1, g6, b6, l2, g7, b7):
    xt0 = _prep(x.transpose(0, 2, 1), disc.transpose(0, 2, 1), pe_w, pe_b)

    x1 = _edge_layer(xt0, w1, r1, para[0], g1, b1)
    x2 = _edge_layer(x1, w2, None, para[2], g2, b2, xx=jnp.sum(x1 * x1, axis=2))
    x3 = _edge_layer(x2, w3, r3, para[4], g3, b3, xx=jnp.sum(x2 * x2, axis=2))
    x4 = _edge_layer(x3, w4, None, para[6], g4, b4, xx=jnp.sum(x3 * x3, axis=2))

    y5 = _y5(x1, x2, x3, x4, w5)
    m5 = jnp.mean(y5, axis=(0, 1))
    v5 = jnp.var(y5, axis=(0, 1))
    mx, sm = _pool(y5, m5, v5, g5, b5)
    return _head(mx, sm, l1, l2, g6, b6, g7, b7)
